# Initial kernel scaffold; baseline (speedup 1.0000x reference)
import jax
import jax.numpy as jnp
from jax.experimental import pallas as pl


def kernel(x, edge_index, edge_weight, W1, b1, W2, b2):
    raise NotImplementedError("write your pallas kernel here")


# speedup vs baseline: 4.0785x; 4.0785x over previous
import functools

import jax
import jax.numpy as jnp
from jax import lax
from jax.experimental import pallas as pl
from jax.experimental.pallas import tpu as pltpu
from jax.experimental.pallas import tpu_sc as plsc

ALPHA = 0.1
KSTEPS = 10

NC = 2
NS = 16
NW = NC * NS
LANES = 16


def _wid():
    return lax.axis_index("c") * NS + lax.axis_index("s")


def _mesh():
    return plsc.VectorSubcoreMesh(core_axis_name="c", subcore_axis_name="s")


_SC_PARAMS = pltpu.CompilerParams(needs_layout_passes=False)


def _deg_kernel(npad, ep):
    ept = ep // NW

    @functools.partial(
        pl.kernel,
        out_type=jax.ShapeDtypeStruct((NW * npad,), jnp.float32),
        mesh=_mesh(),
        compiler_params=_SC_PARAMS,
        scratch_types=[
            pltpu.VMEM((npad,), jnp.float32),
            pltpu.VMEM((ept,), jnp.int32),
            pltpu.VMEM((ept,), jnp.float32),
        ],
    )
    def deg_kernel(col_hbm, w_hbm, part_hbm, deg_t, col_t, w_t):
        wid = _wid()

        def zero_body(i, _):
            deg_t[pl.ds(i * LANES, LANES)] = jnp.zeros((LANES,), jnp.float32)
            return _

        lax.fori_loop(0, npad // LANES, zero_body, None)
        base = wid * ept
        pltpu.sync_copy(col_hbm.at[pl.ds(base, ept)], col_t)
        pltpu.sync_copy(w_hbm.at[pl.ds(base, ept)], w_t)

        def grp(i, _):
            c = col_t[pl.ds(i * LANES, LANES)]
            wv = w_t[pl.ds(i * LANES, LANES)]
            plsc.addupdate_scatter(deg_t, [c], wv)
            return _

        lax.fori_loop(0, ept // LANES, grp, None)
        pltpu.sync_copy(deg_t, part_hbm.at[pl.ds(wid * npad, npad)])

    return deg_kernel


def _dinv_kernel(npad):
    npt = npad // NW

    @functools.partial(
        pl.kernel,
        out_type=jax.ShapeDtypeStruct((npad,), jnp.float32),
        mesh=_mesh(),
        compiler_params=_SC_PARAMS,
        scratch_types=[
            pltpu.VMEM((NW * npt,), jnp.float32),
            pltpu.VMEM((npt,), jnp.float32),
        ],
    )
    def dinv_kernel(part_hbm, dinv_hbm, part_t, dinv_t):
        wid = _wid()
        nbase = wid * npt
        for r in range(NW):
            pltpu.sync_copy(part_hbm.at[pl.ds(r * npad + nbase, npt)],
                            part_t.at[pl.ds(r * npt, npt)])

        def body(i, _):
            acc = part_t[pl.ds(i * LANES, LANES)]
            for r in range(1, NW):
                acc = acc + part_t[pl.ds(r * npt + i * LANES, LANES)]
            bits = plsc.bitcast(acc, jnp.int32)
            y = plsc.bitcast(
                jnp.int32(0x5F3759DF) - lax.shift_right_arithmetic(bits, 1),
                jnp.float32,
            )
            for _i in range(3):
                y = y * (1.5 - 0.5 * acc * y * y)
            dinv_t[pl.ds(i * LANES, LANES)] = jnp.where(acc > 0.0, y, 0.0)
            return _

        lax.fori_loop(0, npt // LANES, body, None)
        pltpu.sync_copy(dinv_t, dinv_hbm.at[pl.ds(nbase, npt)])

    return dinv_kernel


def _norm_kernel(npad, ep):
    ept = ep // NW

    @functools.partial(
        pl.kernel,
        out_type=[
            jax.ShapeDtypeStruct((ep,), jnp.int32),
            jax.ShapeDtypeStruct((ep,), jnp.float32),
        ],
        mesh=_mesh(),
        compiler_params=_SC_PARAMS,
        scratch_types=[
            pltpu.VMEM((npad,), jnp.float32),
            pltpu.VMEM((ept,), jnp.int32),
            pltpu.VMEM((ept,), jnp.int32),
            pltpu.VMEM((ept,), jnp.float32),
            pltpu.VMEM((ept,), jnp.int32),
            pltpu.VMEM((ept,), jnp.float32),
        ],
    )
    def norm_kernel(row_hbm, col_hbm, w_hbm, dinv_hbm, rc_hbm, nm_hbm,
                    dinv_t, row_t, col_t, w_t, rc_t, nm_t):
        wid = _wid()
        base = wid * ept
        pltpu.sync_copy(dinv_hbm, dinv_t)
        pltpu.sync_copy(row_hbm.at[pl.ds(base, ept)], row_t)
        pltpu.sync_copy(col_hbm.at[pl.ds(base, ept)], col_t)
        pltpu.sync_copy(w_hbm.at[pl.ds(base, ept)], w_t)

        def grp(i, _):
            sl = pl.ds(i * LANES, LANES)
            r = row_t[sl]
            c = col_t[sl]
            wv = w_t[sl]
            dr = plsc.load_gather(dinv_t, [r])
            dc = plsc.load_gather(dinv_t, [c])
            nm_t[sl] = dr * wv * dc
            rc_t[sl] = jnp.bitwise_or(r, lax.shift_left(c, 16))
            return _

        lax.fori_loop(0, ept // LANES, grp, None)
        pltpu.sync_copy(rc_t, rc_hbm.at[pl.ds(base, ept)])
        pltpu.sync_copy(nm_t, nm_hbm.at[pl.ds(base, ept)])

    return norm_kernel


def _prop_kernel(npad, ep, fpt, chunk):
    nchunks = ep // chunk
    assert nchunks % 2 == 0

    @functools.partial(
        pl.kernel,
        out_type=jax.ShapeDtypeStruct((NW * fpt * npad,), jnp.float32),
        mesh=_mesh(),
        compiler_params=_SC_PARAMS,
        scratch_types=[
            pltpu.VMEM((fpt * npad,), jnp.float32),
            pltpu.VMEM((fpt * npad,), jnp.float32),
            pltpu.VMEM((fpt * npad,), jnp.float32),
            pltpu.VMEM((chunk,), jnp.int32),
            pltpu.VMEM((chunk,), jnp.int32),
            pltpu.VMEM((chunk,), jnp.float32),
            pltpu.VMEM((chunk,), jnp.float32),
            pltpu.SemaphoreType.DMA,
            pltpu.SemaphoreType.DMA,
        ],
    )
    def prop_kernel(ht_hbm, rc_hbm, nm_hbm, out_hbm,
                    out_t, agg_t, hp_t, rc0, rc1, nm0, nm1, sem0, sem1):
        wid = _wid()
        fbase = wid * fpt * npad
        rcb = (rc0, rc1)
        nmb = (nm0, nm1)
        sems = (sem0, sem1)
        fvec = [jnp.full((LANES,), f * npad, jnp.int32) for f in range(fpt)]

        pltpu.sync_copy(ht_hbm.at[pl.ds(fbase, fpt * npad)], out_t)

        def init_body(i, _):
            sl = pl.ds(i * LANES, LANES)
            hp_t[sl] = out_t[sl] * ALPHA
            agg_t[sl] = jnp.zeros((LANES,), jnp.float32)
            return _

        lax.fori_loop(0, fpt * npad // LANES, init_body, None)

        def k_body(_k, __):
            for b in range(2):
                pltpu.async_copy(rc_hbm.at[pl.ds(b * chunk, chunk)],
                                 rcb[b], sems[b])
                pltpu.async_copy(nm_hbm.at[pl.ds(b * chunk, chunk)],
                                 nmb[b], sems[b])

            def chunk_body(g2, _):
                for b in range(2):
                    g = g2 * 2 + b
                    pltpu.make_async_copy(rc_hbm.at[pl.ds(0, chunk)],
                                          rcb[b], sems[b]).wait()
                    pltpu.make_async_copy(nm_hbm.at[pl.ds(0, chunk)],
                                          nmb[b], sems[b]).wait()

                    def grp(j, _):
                        sl = pl.ds(j * LANES, LANES)
                        rcv = rcb[b][sl]
                        nmv = nmb[b][sl]
                        r = jnp.bitwise_and(rcv, 0xFFFF)
                        c = lax.shift_right_logical(rcv, 16)
                        for f in range(fpt):
                            gv = plsc.load_gather(out_t, [fvec[f] + r])
                            plsc.addupdate_scatter(
                                agg_t, [fvec[f] + c], gv * nmv)
                        return _

                    lax.fori_loop(0, chunk // LANES, grp, None)

                    @pl.when(g2 < nchunks // 2 - 1)
                    def _prefetch():
                        nbase = (g + 2) * chunk
                        pltpu.async_copy(rc_hbm.at[pl.ds(nbase, chunk)],
                                         rcb[b], sems[b])
                        pltpu.async_copy(nm_hbm.at[pl.ds(nbase, chunk)],
                                         nmb[b], sems[b])
                return _

            lax.fori_loop(0, nchunks // 2, chunk_body, None)

            def comb(i, _):
                sl = pl.ds(i * LANES, LANES)
                out_t[sl] = agg_t[sl] * (1.0 - ALPHA) + hp_t[sl]
                agg_t[sl] = jnp.zeros((LANES,), jnp.float32)
                return _

            lax.fori_loop(0, fpt * npad // LANES, comb, None)
            return __

        lax.fori_loop(0, KSTEPS, k_body, None)
        pltpu.sync_copy(out_t, out_hbm.at[pl.ds(fbase, fpt * npad)])

    return prop_kernel


def _mlp_kernel(npad, in_ch, hid_ch, out_ch, bn):
    def body(x_ref, w1_ref, b1_ref, w2_ref, b2_ref, ht_ref):
        t = lax.dot_general(w1_ref[...], x_ref[...],
                            (((1,), (1,)), ((), ())),
                            preferred_element_type=jnp.float32)
        t = jnp.maximum(t + b1_ref[...], 0.0)
        h = lax.dot_general(w2_ref[...], t,
                            (((1,), (0,)), ((), ())),
                            preferred_element_type=jnp.float32)
        ht_ref[...] = h + b2_ref[...]

    grid = (npad // bn,)
    return pl.pallas_call(
        body,
        grid=grid,
        in_specs=[
            pl.BlockSpec((bn, in_ch), lambda i: (i, 0)),
            pl.BlockSpec((hid_ch, in_ch), lambda i: (0, 0)),
            pl.BlockSpec((hid_ch, 1), lambda i: (0, 0)),
            pl.BlockSpec((out_ch, hid_ch), lambda i: (0, 0)),
            pl.BlockSpec((out_ch, 1), lambda i: (0, 0)),
        ],
        out_specs=pl.BlockSpec((out_ch, bn), lambda i: (0, i)),
        out_shape=jax.ShapeDtypeStruct((out_ch, npad), jnp.float32),
    )


def kernel(x, edge_index, edge_weight, W1, b1, W2, b2):
    n, in_ch = x.shape
    hid_ch = W1.shape[0]
    out_ch = W2.shape[0]
    e = edge_index.shape[1]

    fpt = out_ch // NW
    npad = ((n + 2047) // 2048) * 2048
    chunk = 1024
    quant = max(2 * chunk, 8 * NW)
    ep = ((e + n + quant - 1) // quant) * quant

    loop = jnp.arange(n, dtype=jnp.int32)
    pad_i = jnp.zeros((ep - e - n,), jnp.int32)
    pad_f = jnp.zeros((ep - e - n,), jnp.float32)
    rowx = jnp.concatenate([edge_index[0], loop, pad_i])
    colx = jnp.concatenate([edge_index[1], loop, pad_i])
    wx = jnp.concatenate([edge_weight, jnp.ones((n,), jnp.float32), pad_f])

    xpad = jnp.pad(x, ((0, npad - n), (0, 0)))

    parts = _deg_kernel(npad, ep)(colx, wx)
    dinv = _dinv_kernel(npad)(parts)
    rc, nm = _norm_kernel(npad, ep)(rowx, colx, wx, dinv)
    ht = _mlp_kernel(npad, in_ch, hid_ch, out_ch, 2048)(
        xpad, W1, b1.reshape(hid_ch, 1), W2, b2.reshape(out_ch, 1))
    outt = _prop_kernel(npad, ep, fpt, chunk)(ht.reshape(-1), rc, nm)
    return outt.reshape(out_ch, npad)[:, :n].T
